# baseline (speedup 1.0000x reference)
import functools

import jax
import jax.numpy as jnp
from jax import lax
from jax.experimental import pallas as pl
from jax.experimental.pallas import tpu as pltpu
from jax.experimental.pallas import tpu_sc as plsc

E = 8
K = 2
D = 1024
F = 2048
N = 2048
BM = 256
MAXB = (K * N) // BM + E
XR = MAXB * BM
SBM = 512
NW = 32


def _router_body(x_ref, gw_ref, eb_ref, probs_ref, pos_ref, w_ref,
                 bexp_ref, nb_ref):
    x = x_ref[...]
    logits = lax.dot_general(x, gw_ref[...], (((1,), (1,)), ((), ())),
                             preferred_element_type=jnp.float32)
    logits = logits + eb_ref[...]
    m = jnp.max(logits, axis=1, keepdims=True)
    ex = jnp.exp(logits - m)
    probs = ex / jnp.sum(ex, axis=1, keepdims=True)
    probs_ref[...] = probs

    col = lax.broadcasted_iota(jnp.int32, (N, E), 1)
    p0 = jnp.max(probs, axis=1, keepdims=True)
    i0 = jnp.min(jnp.where(probs == p0, col, E), axis=1, keepdims=True)
    pm = jnp.where(col == i0, -jnp.inf, probs)
    p1 = jnp.max(pm, axis=1, keepdims=True)
    i1 = jnp.min(jnp.where(pm == p1, col, E), axis=1, keepdims=True)

    denom = p0 + p1
    w_ref[...] = jnp.concatenate([p0 / denom, p1 / denom], axis=1)

    oh0 = (col == i0).astype(jnp.float32)
    oh1 = (col == i1).astype(jnp.float32)

    CB = 256
    r = lax.broadcasted_iota(jnp.int32, (CB, CB), 0)
    c = lax.broadcasted_iota(jnp.int32, (CB, CB), 1)
    tri = (c < r).astype(jnp.float32)

    def excl_cumsum(oh):
        carry = jnp.zeros((1, E), jnp.float32)
        parts = []
        for j in range(N // CB):
            blk = lax.slice(oh, (j * CB, 0), ((j + 1) * CB, E))
            parts.append(lax.dot_general(tri, blk, (((1,), (0,)), ((), ())),
                                         preferred_element_type=jnp.float32)
                         + carry)
            carry = carry + jnp.sum(blk, axis=0, keepdims=True)
        return jnp.concatenate(parts, axis=0), carry

    r0, c0 = excl_cumsum(oh0)
    r1, c1 = excl_cumsum(oh1)
    rank0 = jnp.sum(r0 * oh0, axis=1, keepdims=True)
    rank1 = jnp.sum(r1 * oh1, axis=1, keepdims=True)
    c0_at_i1 = jnp.sum(oh1 * c0, axis=1, keepdims=True)

    counts = c0 + c1
    padded = jnp.ceil(counts / BM) * BM
    er = lax.broadcasted_iota(jnp.int32, (E, E), 0)
    ec = lax.broadcasted_iota(jnp.int32, (E, E), 1)
    upper = (er < ec).astype(jnp.float32)
    aligned_off = lax.dot_general(padded, upper, (((1,), (0,)), ((), ())),
                                  preferred_element_type=jnp.float32)
    off_at_i0 = jnp.sum(oh0 * aligned_off, axis=1, keepdims=True)
    off_at_i1 = jnp.sum(oh1 * aligned_off, axis=1, keepdims=True)
    pos0 = off_at_i0.astype(jnp.int32) + rank0.astype(jnp.int32)
    pos1 = (off_at_i1 + c0_at_i1).astype(jnp.int32) + rank1.astype(jnp.int32)
    pos_ref[...] = jnp.concatenate([pos0, pos1], axis=1)

    ends = aligned_off + padded
    brow = (lax.broadcasted_iota(jnp.int32, (MAXB, E), 0) * BM).astype(jnp.float32)
    bexp = jnp.sum((brow >= ends).astype(jnp.int32), axis=1, keepdims=True)
    bexp_ref[...] = jnp.minimum(bexp, E - 1)
    nb_ref[...] = (jnp.sum(padded, axis=1, keepdims=True) / BM).astype(jnp.int32)


def _router(x, gate_w, expert_bias):
    return pl.pallas_call(
        _router_body,
        out_shape=(
            jax.ShapeDtypeStruct((N, E), jnp.float32),
            jax.ShapeDtypeStruct((N, K), jnp.int32),
            jax.ShapeDtypeStruct((N, K), jnp.float32),
            jax.ShapeDtypeStruct((MAXB, 1), jnp.int32),
            jax.ShapeDtypeStruct((1, 1), jnp.int32),
        ),
    )(x, gate_w, expert_bias)


def _dispatch_body(x_hbm, pos_hbm, xs_hbm, idx_v, rows_v, osem0, osem1):
    wid = lax.axis_index("s") * 2 + lax.axis_index("c")
    tok_base = (wid % 16) * 128
    pltpu.sync_copy(pos_hbm.at[wid], idx_v)
    osems = (osem0, osem1)

    def wait_scatter(jj):
        pltpu.make_async_copy(rows_v.at[jj], xs_hbm.at[idx_v[jj]],
                              osems[jj]).wait()

    for j in range(8):
        jj = j % 2
        if j >= 2:
            wait_scatter(jj)
        pltpu.sync_copy(x_hbm.at[pl.ds(tok_base + j * 16, 16)], rows_v.at[jj])
        pltpu.make_async_copy(rows_v.at[jj], xs_hbm.at[idx_v[j]],
                              osems[jj]).start()
    wait_scatter(0)
    wait_scatter(1)


def _dispatch(x_i32, pos_disp):
    mesh = plsc.VectorSubcoreMesh(core_axis_name="c", subcore_axis_name="s")
    return pl.kernel(
        _dispatch_body,
        out_type=jax.ShapeDtypeStruct((XR, D // 2), jnp.int32),
        mesh=mesh,
        scratch_types=[
            pltpu.VMEM((8, 16), jnp.int32),
            pltpu.VMEM((2, 16, D // 2), jnp.int32),
            pltpu.SemaphoreType.DMA,
            pltpu.SemaphoreType.DMA,
        ],
    )(x_i32, pos_disp)


def _ffn_compute(x_f32, w1b, w3b, w2b):
    xb = x_f32.astype(jnp.bfloat16)
    h1 = lax.dot_general(xb, w1b, (((1,), (1,)), ((), ())),
                         preferred_element_type=jnp.float32)
    h3 = lax.dot_general(xb, w3b, (((1,), (1,)), ((), ())),
                         preferred_element_type=jnp.float32)
    h = (h1 * jax.nn.sigmoid(h1)) * h3
    return lax.dot_general(h.astype(jnp.bfloat16), w2b, (((1,), (1,)), ((), ())),
                           preferred_element_type=jnp.float32)


def _shared_body(x_ref, w1a_ref, w1c_ref, w3a_ref, w3c_ref, w2a_ref, w2c_ref,
                 o_ref, w1b, w3b, w2b):
    FH = F // 2
    @pl.when(pl.program_id(0) == 0)
    def _():
        w1b[:FH] = w1a_ref[...].astype(jnp.bfloat16)
        w1b[FH:] = w1c_ref[...].astype(jnp.bfloat16)
        w3b[:FH] = w3a_ref[...].astype(jnp.bfloat16)
        w3b[FH:] = w3c_ref[...].astype(jnp.bfloat16)
        w2b[:, :FH] = w2a_ref[...].astype(jnp.bfloat16)
        w2b[:, FH:] = w2c_ref[...].astype(jnp.bfloat16)
    o_ref[...] = _ffn_compute(x_ref[...], w1b[...], w3b[...], w2b[...])


def _shared(x, sw1, sw3, sw2):
    FH = F // 2
    return pl.pallas_call(
        _shared_body,
        grid=(N // SBM,),
        in_specs=[
            pl.BlockSpec((SBM, D), lambda b: (b, 0)),
            pl.BlockSpec((FH, D), lambda b: (0, 0)),
            pl.BlockSpec((FH, D), lambda b: (1, 0)),
            pl.BlockSpec((FH, D), lambda b: (0, 0)),
            pl.BlockSpec((FH, D), lambda b: (1, 0)),
            pl.BlockSpec((D, FH), lambda b: (0, 0)),
            pl.BlockSpec((D, FH), lambda b: (0, 1)),
        ],
        out_specs=pl.BlockSpec((SBM, D), lambda b: (b, 0)),
        out_shape=jax.ShapeDtypeStruct((N, D), jnp.float32),
        scratch_shapes=[
            pltpu.VMEM((F, D), jnp.bfloat16),
            pltpu.VMEM((F, D), jnp.bfloat16),
            pltpu.VMEM((D, F), jnp.bfloat16),
        ],
        compiler_params=pltpu.CompilerParams(
            vmem_limit_bytes=100 * 1024 * 1024),
    )(x, sw1, sw1, sw3, sw3, sw2, sw2)


def _routed_body(bexp_ref, nb_ref, xs_ref, w1a_ref, w1c_ref, w3a_ref, w3c_ref,
                 w2_ref, ys_ref, w1b, w3b, w2b, w2f, sem0, sem1):
    FH = F // 2
    b = pl.program_id(0)
    nbv = nb_ref[0]
    active = b < nbv
    e = bexp_ref[b]
    e_prev = bexp_ref[lax.max(b - 1, 0)]
    e_next = bexp_ref[lax.min(b + 1, MAXB - 1)]

    def start_w2(ee):
        pltpu.make_async_copy(w2_ref.at[ee, slice(None), pl.ds(0, FH)],
                              w2f.at[:, pl.ds(0, FH)], sem0).start()
        pltpu.make_async_copy(w2_ref.at[ee, slice(None), pl.ds(FH, FH)],
                              w2f.at[:, pl.ds(FH, FH)], sem1).start()

    def wait_w2():
        pltpu.make_async_copy(w2_ref.at[0, slice(None), pl.ds(0, FH)],
                              w2f.at[:, pl.ds(0, FH)], sem0).wait()
        pltpu.make_async_copy(w2_ref.at[0, slice(None), pl.ds(FH, FH)],
                              w2f.at[:, pl.ds(FH, FH)], sem1).wait()

    @pl.when(active & (b == 0))
    def _():
        start_w2(e)

    @pl.when(active & ((b == 0) | (e != e_prev)))
    def _():
        w1b[:FH] = w1a_ref[0].astype(jnp.bfloat16)
        w1b[FH:] = w1c_ref[0].astype(jnp.bfloat16)
        w3b[:FH] = w3a_ref[0].astype(jnp.bfloat16)
        w3b[FH:] = w3c_ref[0].astype(jnp.bfloat16)
        wait_w2()
        w2b[...] = w2f[...].astype(jnp.bfloat16)

    @pl.when(active & (e_next != e) & (b + 1 < nbv))
    def _():
        start_w2(e_next)

    @pl.when(active)
    def _():
        ys_ref[...] = _ffn_compute(xs_ref[...], w1b[...], w3b[...], w2b[...])


def _routed(bexp, nb, xs, rw1, rw3, rw2):
    FH = F // 2

    def row_map(b, bexp_ref, nb_ref):
        return (lax.min(b, nb_ref[0] - 1), 0)

    def wa_map(b, bexp_ref, nb_ref):
        return (bexp_ref[b], 0, 0)

    def wc_map(b, bexp_ref, nb_ref):
        return (bexp_ref[b], 1, 0)

    grid_spec = pltpu.PrefetchScalarGridSpec(
        num_scalar_prefetch=2,
        grid=(MAXB,),
        in_specs=[
            pl.BlockSpec((BM, D), row_map),
            pl.BlockSpec((1, FH, D), wa_map),
            pl.BlockSpec((1, FH, D), wc_map),
            pl.BlockSpec((1, FH, D), wa_map),
            pl.BlockSpec((1, FH, D), wc_map),
            pl.BlockSpec(memory_space=pl.ANY),
        ],
        out_specs=pl.BlockSpec((BM, D), row_map),
        scratch_shapes=[
            pltpu.VMEM((F, D), jnp.bfloat16),
            pltpu.VMEM((F, D), jnp.bfloat16),
            pltpu.VMEM((D, F), jnp.bfloat16),
            pltpu.VMEM((D, F), jnp.float32),
            pltpu.SemaphoreType.DMA,
            pltpu.SemaphoreType.DMA,
        ],
    )
    return pl.pallas_call(
        _routed_body,
        grid_spec=grid_spec,
        out_shape=jax.ShapeDtypeStruct((XR, D), jnp.float32),
        compiler_params=pltpu.CompilerParams(
            vmem_limit_bytes=100 * 1024 * 1024),
    )(bexp, nb, xs, rw1, rw1, rw3, rw3, rw2)


def _combine_body(ys_hbm, sh_hbm, pos_hbm, wsp_hbm, out_hbm,
                  idx_v, wsp_v, rr_v, sh_v, acc_v,
                  gsem0, gsem1, osem0):
    wid = lax.axis_index("s") * 2 + lax.axis_index("c")
    pltpu.sync_copy(pos_hbm.at[wid], idx_v)
    pltpu.sync_copy(wsp_hbm.at[wid], wsp_v)
    gsems = (gsem0, gsem1)

    def start_in(j):
        jj = j % 2
        pltpu.make_async_copy(ys_hbm.at[idx_v.at[j]], rr_v.at[jj],
                              gsems[jj]).start()

    def wait_out():
        pltpu.make_async_copy(acc_v, out_hbm.at[pl.ds(0, 16)], osem0).wait()

    start_in(0)
    for j in range(4):
        jj = j % 2
        if j + 1 < 4:
            start_in(j + 1)
        pltpu.sync_copy(sh_hbm.at[pl.ds(wid * 64 + j * 16, 16)], sh_v)
        pltpu.make_async_copy(ys_hbm.at[idx_v.at[j]], rr_v.at[jj],
                              gsems[jj]).wait()
        if j >= 1:
            wait_out()
        w0s = [wsp_v[0, j, i, :] for i in range(16)]
        w1s = [wsp_v[1, j, i, :] for i in range(16)]

        def body(cc, carry):
            sl = pl.ds(pl.multiple_of(cc * 16, 16), 16)
            for i in range(16):
                acc_v[i, sl] = (sh_v[i, sl]
                                + w0s[i] * rr_v[jj, i, sl]
                                + w1s[i] * rr_v[jj, 16 + i, sl])
            return carry

        lax.fori_loop(0, D // 16, body, 0)
        pltpu.make_async_copy(acc_v,
                              out_hbm.at[pl.ds(wid * 64 + j * 16, 16)],
                              osem0).start()
    wait_out()


def _combine(ys, shared, pos_c, w_c):
    mesh = plsc.VectorSubcoreMesh(core_axis_name="c", subcore_axis_name="s")
    return pl.kernel(
        _combine_body,
        out_type=jax.ShapeDtypeStruct((N, D), jnp.float32),
        mesh=mesh,
        scratch_types=[
            pltpu.VMEM((4, 32), jnp.int32),
            pltpu.VMEM((2, 4, 16, 16), jnp.float32),
            pltpu.VMEM((2, 32, D), jnp.float32),
            pltpu.VMEM((16, D), jnp.float32),
            pltpu.VMEM((16, D), jnp.float32),
            pltpu.SemaphoreType.DMA,
            pltpu.SemaphoreType.DMA,
            pltpu.SemaphoreType.DMA,
        ],
    )(ys, shared, pos_c, w_c)


def kernel(x, gate_w, sw1, sw2, sw3, rw1, rw2, rw3, expert_bias):
    Bx, Lx, Dx = x.shape
    xf = x.reshape(N, D)
    probs, pos_tk, w_tk, bexp2, nb2 = _router(xf, gate_w,
                                              expert_bias.reshape(1, E))
    bexp = bexp2.reshape(MAXB)
    nb = nb2.reshape(1)
    pos_km = pos_tk.T
    pos_disp = pos_km.reshape(NW, 8, 16)
    pos_c = jnp.concatenate([pos_km[0].reshape(NW, 4, 16),
                             pos_km[1].reshape(NW, 4, 16)],
                            axis=2)
    w_c = (jnp.broadcast_to(w_tk.T[:, :, None], (K, N, 16))
           .reshape(K, NW, 4, 16, 16).transpose(1, 0, 2, 3, 4))

    xbf = xf.astype(jnp.bfloat16)
    x_i32 = lax.bitcast_convert_type(xbf.reshape(N, D // 2, 2), jnp.int32)
    xs_i32 = _dispatch(x_i32, pos_disp)
    xs_bf = lax.bitcast_convert_type(xs_i32, jnp.bfloat16).reshape(XR, D)
    shared = _shared(xbf, sw1, sw3, sw2)
    ys = _routed(bexp, nb, xs_bf, rw1, rw3, rw2)
    out = _combine(ys, shared, pos_c, w_c)
    return out.reshape(Bx, Lx, Dx), probs


# speedup vs baseline: 1.7576x; 1.7576x over previous
import functools

import jax
import jax.numpy as jnp
from jax import lax
from jax.experimental import pallas as pl
from jax.experimental.pallas import tpu as pltpu
from jax.experimental.pallas import tpu_sc as plsc

E = 8
K = 2
D = 1024
F = 2048
N = 2048
BM = 256
MAXB = (K * N) // BM + E
XR = MAXB * BM
SBM = 512
NW = 32


def _router_body(x_ref, gw_ref, eb_ref, probs_ref, pos_ref, w_ref,
                 bexp_ref, nb_ref):
    x = x_ref[...]
    logits = lax.dot_general(x, gw_ref[...], (((1,), (1,)), ((), ())),
                             preferred_element_type=jnp.float32)
    logits = logits + eb_ref[...]
    m = jnp.max(logits, axis=1, keepdims=True)
    ex = jnp.exp(logits - m)
    probs = ex / jnp.sum(ex, axis=1, keepdims=True)
    probs_ref[...] = probs

    col = lax.broadcasted_iota(jnp.int32, (N, E), 1)
    p0 = jnp.max(probs, axis=1, keepdims=True)
    i0 = jnp.min(jnp.where(probs == p0, col, E), axis=1, keepdims=True)
    pm = jnp.where(col == i0, -jnp.inf, probs)
    p1 = jnp.max(pm, axis=1, keepdims=True)
    i1 = jnp.min(jnp.where(pm == p1, col, E), axis=1, keepdims=True)

    denom = p0 + p1
    w_ref[...] = jnp.concatenate([p0 / denom, p1 / denom], axis=1)

    oh0 = (col == i0).astype(jnp.float32)
    oh1 = (col == i1).astype(jnp.float32)

    CB = 256
    r = lax.broadcasted_iota(jnp.int32, (CB, CB), 0)
    c = lax.broadcasted_iota(jnp.int32, (CB, CB), 1)
    tri = (c < r).astype(jnp.float32)

    def excl_cumsum(oh):
        carry = jnp.zeros((1, E), jnp.float32)
        parts = []
        for j in range(N // CB):
            blk = lax.slice(oh, (j * CB, 0), ((j + 1) * CB, E))
            parts.append(lax.dot_general(tri, blk, (((1,), (0,)), ((), ())),
                                         preferred_element_type=jnp.float32)
                         + carry)
            carry = carry + jnp.sum(blk, axis=0, keepdims=True)
        return jnp.concatenate(parts, axis=0), carry

    r0, c0 = excl_cumsum(oh0)
    r1, c1 = excl_cumsum(oh1)
    rank0 = jnp.sum(r0 * oh0, axis=1, keepdims=True)
    rank1 = jnp.sum(r1 * oh1, axis=1, keepdims=True)
    c0_at_i1 = jnp.sum(oh1 * c0, axis=1, keepdims=True)

    counts = c0 + c1
    padded = jnp.ceil(counts / BM) * BM
    er = lax.broadcasted_iota(jnp.int32, (E, E), 0)
    ec = lax.broadcasted_iota(jnp.int32, (E, E), 1)
    upper = (er < ec).astype(jnp.float32)
    aligned_off = lax.dot_general(padded, upper, (((1,), (0,)), ((), ())),
                                  preferred_element_type=jnp.float32)
    off_at_i0 = jnp.sum(oh0 * aligned_off, axis=1, keepdims=True)
    off_at_i1 = jnp.sum(oh1 * aligned_off, axis=1, keepdims=True)
    pos0 = off_at_i0.astype(jnp.int32) + rank0.astype(jnp.int32)
    pos1 = (off_at_i1 + c0_at_i1).astype(jnp.int32) + rank1.astype(jnp.int32)
    pos_ref[...] = jnp.concatenate([pos0, pos1], axis=1)

    ends = aligned_off + padded
    brow = (lax.broadcasted_iota(jnp.int32, (MAXB, E), 0) * BM).astype(jnp.float32)
    bexp = jnp.sum((brow >= ends).astype(jnp.int32), axis=1, keepdims=True)
    bexp_ref[...] = jnp.minimum(bexp, E - 1)
    nb_ref[...] = (jnp.sum(padded, axis=1, keepdims=True) / BM).astype(jnp.int32)


def _router(x, gate_w, expert_bias):
    return pl.pallas_call(
        _router_body,
        out_shape=(
            jax.ShapeDtypeStruct((N, E), jnp.float32),
            jax.ShapeDtypeStruct((N, K), jnp.int32),
            jax.ShapeDtypeStruct((N, K), jnp.float32),
            jax.ShapeDtypeStruct((MAXB, 1), jnp.int32),
            jax.ShapeDtypeStruct((1, 1), jnp.int32),
        ),
    )(x, gate_w, expert_bias)


def _dispatch_body(x_hbm, pos_hbm, xs_hbm, idx_v, rows_v, osem0, osem1):
    wid = lax.axis_index("s") * 2 + lax.axis_index("c")
    tok_base = (wid % 16) * 128
    pltpu.sync_copy(pos_hbm.at[wid], idx_v)
    osems = (osem0, osem1)

    def wait_scatter(jj):
        pltpu.make_async_copy(rows_v.at[jj], xs_hbm.at[idx_v[jj]],
                              osems[jj]).wait()

    for j in range(8):
        jj = j % 2
        if j >= 2:
            wait_scatter(jj)
        pltpu.sync_copy(x_hbm.at[pl.ds(tok_base + j * 16, 16)], rows_v.at[jj])
        pltpu.make_async_copy(rows_v.at[jj], xs_hbm.at[idx_v[j]],
                              osems[jj]).start()
    wait_scatter(0)
    wait_scatter(1)


def _dispatch(x, pos_disp):
    mesh = plsc.VectorSubcoreMesh(core_axis_name="c", subcore_axis_name="s")
    return pl.kernel(
        _dispatch_body,
        out_type=jax.ShapeDtypeStruct((XR, D), jnp.float32),
        mesh=mesh,
        scratch_types=[
            pltpu.VMEM((8, 16), jnp.int32),
            pltpu.VMEM((2, 16, D), jnp.float32),
            pltpu.SemaphoreType.DMA,
            pltpu.SemaphoreType.DMA,
        ],
    )(x, pos_disp)


def _ffn_compute(x_f32, w1b, w3b, w2b):
    xb = x_f32.astype(jnp.bfloat16)
    h1 = lax.dot_general(xb, w1b, (((1,), (1,)), ((), ())),
                         preferred_element_type=jnp.float32)
    h3 = lax.dot_general(xb, w3b, (((1,), (1,)), ((), ())),
                         preferred_element_type=jnp.float32)
    h = (h1 * jax.nn.sigmoid(h1)) * h3
    return lax.dot_general(h.astype(jnp.bfloat16), w2b, (((1,), (1,)), ((), ())),
                           preferred_element_type=jnp.float32)


def _shared_body(x_ref, w1a_ref, w1c_ref, w3a_ref, w3c_ref, w2a_ref, w2c_ref,
                 o_ref, w1b, w3b, w2b):
    FH = F // 2
    @pl.when(pl.program_id(0) == 0)
    def _():
        w1b[:FH] = w1a_ref[...].astype(jnp.bfloat16)
        w1b[FH:] = w1c_ref[...].astype(jnp.bfloat16)
        w3b[:FH] = w3a_ref[...].astype(jnp.bfloat16)
        w3b[FH:] = w3c_ref[...].astype(jnp.bfloat16)
        w2b[:, :FH] = w2a_ref[...].astype(jnp.bfloat16)
        w2b[:, FH:] = w2c_ref[...].astype(jnp.bfloat16)
    o_ref[...] = _ffn_compute(x_ref[...], w1b[...], w3b[...], w2b[...])


def _shared(x, sw1, sw3, sw2):
    FH = F // 2
    return pl.pallas_call(
        _shared_body,
        grid=(N // SBM,),
        in_specs=[
            pl.BlockSpec((SBM, D), lambda b: (b, 0)),
            pl.BlockSpec((FH, D), lambda b: (0, 0)),
            pl.BlockSpec((FH, D), lambda b: (1, 0)),
            pl.BlockSpec((FH, D), lambda b: (0, 0)),
            pl.BlockSpec((FH, D), lambda b: (1, 0)),
            pl.BlockSpec((D, FH), lambda b: (0, 0)),
            pl.BlockSpec((D, FH), lambda b: (0, 1)),
        ],
        out_specs=pl.BlockSpec((SBM, D), lambda b: (b, 0)),
        out_shape=jax.ShapeDtypeStruct((N, D), jnp.float32),
        scratch_shapes=[
            pltpu.VMEM((F, D), jnp.bfloat16),
            pltpu.VMEM((F, D), jnp.bfloat16),
            pltpu.VMEM((D, F), jnp.bfloat16),
        ],
        compiler_params=pltpu.CompilerParams(
            vmem_limit_bytes=100 * 1024 * 1024),
    )(x, sw1, sw1, sw3, sw3, sw2, sw2)


def _routed_body(bexp_ref, nb_ref, xs_ref, w1a_ref, w1c_ref, w3a_ref, w3c_ref,
                 w2_ref, ys_ref, w1b, w3b, w2b, w2f, sem0, sem1):
    FH = F // 2
    b = pl.program_id(0)
    nbv = nb_ref[0]
    active = b < nbv
    e = bexp_ref[b]
    e_prev = bexp_ref[lax.max(b - 1, 0)]
    e_next = bexp_ref[lax.min(b + 1, MAXB - 1)]

    def start_w2(ee):
        pltpu.make_async_copy(w2_ref.at[ee, slice(None), pl.ds(0, FH)],
                              w2f.at[:, pl.ds(0, FH)], sem0).start()
        pltpu.make_async_copy(w2_ref.at[ee, slice(None), pl.ds(FH, FH)],
                              w2f.at[:, pl.ds(FH, FH)], sem1).start()

    def wait_w2():
        pltpu.make_async_copy(w2_ref.at[0, slice(None), pl.ds(0, FH)],
                              w2f.at[:, pl.ds(0, FH)], sem0).wait()
        pltpu.make_async_copy(w2_ref.at[0, slice(None), pl.ds(FH, FH)],
                              w2f.at[:, pl.ds(FH, FH)], sem1).wait()

    @pl.when(active & (b == 0))
    def _():
        start_w2(e)

    @pl.when(active & ((b == 0) | (e != e_prev)))
    def _():
        w1b[:FH] = w1a_ref[0].astype(jnp.bfloat16)
        w1b[FH:] = w1c_ref[0].astype(jnp.bfloat16)
        w3b[:FH] = w3a_ref[0].astype(jnp.bfloat16)
        w3b[FH:] = w3c_ref[0].astype(jnp.bfloat16)
        wait_w2()
        w2b[...] = w2f[...].astype(jnp.bfloat16)

    @pl.when(active & (e_next != e) & (b + 1 < nbv))
    def _():
        start_w2(e_next)

    @pl.when(active)
    def _():
        ys_ref[...] = _ffn_compute(xs_ref[...], w1b[...], w3b[...], w2b[...])


def _routed(bexp, nb, xs, rw1, rw3, rw2):
    FH = F // 2

    def row_map(b, bexp_ref, nb_ref):
        return (lax.min(b, nb_ref[0] - 1), 0)

    def wa_map(b, bexp_ref, nb_ref):
        return (bexp_ref[b], 0, 0)

    def wc_map(b, bexp_ref, nb_ref):
        return (bexp_ref[b], 1, 0)

    grid_spec = pltpu.PrefetchScalarGridSpec(
        num_scalar_prefetch=2,
        grid=(MAXB,),
        in_specs=[
            pl.BlockSpec((BM, D), row_map),
            pl.BlockSpec((1, FH, D), wa_map),
            pl.BlockSpec((1, FH, D), wc_map),
            pl.BlockSpec((1, FH, D), wa_map),
            pl.BlockSpec((1, FH, D), wc_map),
            pl.BlockSpec(memory_space=pl.ANY),
        ],
        out_specs=pl.BlockSpec((BM, D), row_map),
        scratch_shapes=[
            pltpu.VMEM((F, D), jnp.bfloat16),
            pltpu.VMEM((F, D), jnp.bfloat16),
            pltpu.VMEM((D, F), jnp.bfloat16),
            pltpu.VMEM((D, F), jnp.float32),
            pltpu.SemaphoreType.DMA,
            pltpu.SemaphoreType.DMA,
        ],
    )
    return pl.pallas_call(
        _routed_body,
        grid_spec=grid_spec,
        out_shape=jax.ShapeDtypeStruct((XR, D), jnp.float32),
        compiler_params=pltpu.CompilerParams(
            vmem_limit_bytes=100 * 1024 * 1024),
    )(bexp, nb, xs, rw1, rw1, rw3, rw3, rw2)


def _combine_body(ys_hbm, sh_hbm, pos_hbm, wsp_hbm, out_hbm,
                  idx_v, wsp_v, rr_v, sh_v, acc_v,
                  gsem0, gsem1, osem0):
    wid = lax.axis_index("s") * 2 + lax.axis_index("c")
    pltpu.sync_copy(pos_hbm.at[wid], idx_v)
    pltpu.sync_copy(wsp_hbm.at[wid], wsp_v)
    gsems = (gsem0, gsem1)

    def start_in(j):
        jj = j % 2
        pltpu.make_async_copy(ys_hbm.at[idx_v.at[j]], rr_v.at[jj],
                              gsems[jj]).start()

    def wait_out():
        pltpu.make_async_copy(acc_v, out_hbm.at[pl.ds(0, 16)], osem0).wait()

    start_in(0)
    for j in range(4):
        jj = j % 2
        if j + 1 < 4:
            start_in(j + 1)
        pltpu.sync_copy(sh_hbm.at[pl.ds(wid * 64 + j * 16, 16)], sh_v)
        pltpu.make_async_copy(ys_hbm.at[idx_v.at[j]], rr_v.at[jj],
                              gsems[jj]).wait()
        if j >= 1:
            wait_out()
        w0s = [wsp_v[0, j, i, :] for i in range(16)]
        w1s = [wsp_v[1, j, i, :] for i in range(16)]

        def body(cc, carry):
            sl = pl.ds(pl.multiple_of(cc * 16, 16), 16)
            for i in range(16):
                acc_v[i, sl] = (sh_v[i, sl]
                                + w0s[i] * rr_v[jj, i, sl]
                                + w1s[i] * rr_v[jj, 16 + i, sl])
            return carry

        lax.fori_loop(0, D // 16, body, 0)
        pltpu.make_async_copy(acc_v,
                              out_hbm.at[pl.ds(wid * 64 + j * 16, 16)],
                              osem0).start()
    wait_out()


def _combine(ys, shared, pos_c, w_c):
    mesh = plsc.VectorSubcoreMesh(core_axis_name="c", subcore_axis_name="s")
    return pl.kernel(
        _combine_body,
        out_type=jax.ShapeDtypeStruct((N, D), jnp.float32),
        mesh=mesh,
        scratch_types=[
            pltpu.VMEM((4, 32), jnp.int32),
            pltpu.VMEM((2, 4, 16, 16), jnp.float32),
            pltpu.VMEM((2, 32, D), jnp.float32),
            pltpu.VMEM((16, D), jnp.float32),
            pltpu.VMEM((16, D), jnp.float32),
            pltpu.SemaphoreType.DMA,
            pltpu.SemaphoreType.DMA,
            pltpu.SemaphoreType.DMA,
        ],
    )(ys, shared, pos_c, w_c)


def kernel(x, gate_w, sw1, sw2, sw3, rw1, rw2, rw3, expert_bias):
    Bx, Lx, Dx = x.shape
    xf = x.reshape(N, D)
    probs, pos_tk, w_tk, bexp2, nb2 = _router(xf, gate_w,
                                              expert_bias.reshape(1, E))
    bexp = bexp2.reshape(MAXB)
    nb = nb2.reshape(1)
    pos_km = pos_tk.T
    pos_disp = pos_km.reshape(NW, 8, 16)
    pos_c = jnp.concatenate([pos_km[0].reshape(NW, 4, 16),
                             pos_km[1].reshape(NW, 4, 16)],
                            axis=2)
    w_c = (jnp.broadcast_to(w_tk.T[:, :, None], (K, N, 16))
           .reshape(K, NW, 4, 16, 16).transpose(1, 0, 2, 3, 4))

    xs = _dispatch(xf, pos_disp)
    shared = _shared(xf, sw1, sw3, sw2)
    ys = _routed(bexp, nb, xs, rw1, rw3, rw2)
    out = _combine(ys, shared, pos_c, w_c)
    return out.reshape(Bx, Lx, Dx), probs
